# baseline (speedup 1.0000x reference)
import functools

import jax
import jax.numpy as jnp
from jax.experimental import pallas as pl
from jax.experimental.pallas import tpu as pltpu
from jax.experimental.pallas import tpu_sc as plsc

_T = 2048
_D = 1024
_E = 8
_K = 2
_F = 4096
_EPS = 1e-5

_BM = 128
_NT = _T * _K // _BM + _E
_P = _NT * _BM
_FB = 512
_NF = _F // _FB

_NEG = jnp.finfo(jnp.float32).min


def _dispatch_body(x_ref, mask_ref, lnw_ref, wr_ref, tril_ref,
                   pos_ref, pos2_ref, p2_ref, tg_ref, act_ref, ng_ref):
    x = x_ref[...]
    var = jnp.mean(x * x, axis=-1, keepdims=True)
    xn = x * jax.lax.rsqrt(var + _EPS) * lnw_ref[...]
    logits = jnp.dot(xn, wr_ref[...], preferred_element_type=jnp.float32)
    mask = mask_ref[...] != 0
    logits = jnp.where(mask, logits, _NEG)

    ii = jax.lax.broadcasted_iota(jnp.int32, logits.shape, 1)
    m1 = jnp.max(logits, axis=-1, keepdims=True)
    i1 = jnp.min(jnp.where(logits == m1, ii, _E), axis=-1, keepdims=True)
    is1 = ii == i1
    l2 = jnp.where(is1, _NEG, logits)
    m2 = jnp.max(l2, axis=-1, keepdims=True)
    i2 = jnp.min(jnp.where(l2 == m2, ii, _E), axis=-1, keepdims=True)
    is2 = ii == i2
    hard = is1 | is2

    z = jnp.exp(logits - m1)
    probs = z / jnp.sum(z, axis=-1, keepdims=True)
    probs = jnp.where(hard & mask, probs, 0.0)

    sel = hard.astype(jnp.bfloat16)
    rank = jnp.dot(tril_ref[...], sel, preferred_element_type=jnp.float32)
    rank = rank.astype(jnp.int32)
    counts = jnp.sum(hard.astype(jnp.float32), axis=0, keepdims=True)
    counts = counts.astype(jnp.int32)
    c_pad = ((counts + _BM - 1) // _BM) * _BM
    ei = jax.lax.broadcasted_iota(jnp.int32, (_E, _E), 0)
    ej = jax.lax.broadcasted_iota(jnp.int32, (_E, _E), 1)
    upper = (ei < ej).astype(jnp.bfloat16)
    off = jnp.dot(c_pad.astype(jnp.bfloat16), upper,
                  preferred_element_type=jnp.float32).astype(jnp.int32)
    ends = off + c_pad

    pos = off + rank
    ti = jax.lax.broadcasted_iota(jnp.int32, logits.shape, 0)
    pos_ref[...] = jnp.where(hard, pos, _P + ti * _E + ii)

    pos1 = jnp.sum(jnp.where(is1, pos, 0), axis=-1, keepdims=True)
    pos2_ = jnp.sum(jnp.where(is2, pos, 0), axis=-1, keepdims=True)
    pos2_ref[...] = jnp.concatenate([pos1, pos2_], axis=-1)
    pb1 = jnp.sum(jnp.where(is1, probs, 0.0), axis=-1, keepdims=True)
    pb2 = jnp.sum(jnp.where(is2, probs, 0.0), axis=-1, keepdims=True)
    p2_ref[...] = jnp.concatenate([pb1, pb2], axis=-1)

    ms = jax.lax.broadcasted_iota(jnp.int32, (_NT, _E), 0) * _BM
    tg = jnp.sum((ms >= ends).astype(jnp.int32), axis=-1, keepdims=True)
    tg = jnp.minimum(tg, _E - 1)
    tg_ref[...] = tg
    mc = jax.lax.broadcasted_iota(jnp.int32, (_NT, 1), 0) * _BM
    act_ref[...] = (mc < ends[:, _E - 1:_E]).astype(jnp.int32)
    prev = jnp.concatenate([jnp.full((1, 1), -1, jnp.int32), tg[:-1]], axis=0)
    ng_ref[...] = (tg != prev).astype(jnp.int32)


def _dispatch(x, mask, ln_w, w_router, tril):
    return pl.pallas_call(
        _dispatch_body,
        out_shape=[
            jax.ShapeDtypeStruct((_T, _E), jnp.int32),
            jax.ShapeDtypeStruct((_T, _K), jnp.int32),
            jax.ShapeDtypeStruct((_T, _K), jnp.float32),
            jax.ShapeDtypeStruct((_NT, 1), jnp.int32),
            jax.ShapeDtypeStruct((_NT, 1), jnp.int32),
            jax.ShapeDtypeStruct((_NT, 1), jnp.int32),
        ],
    )(x, mask.astype(jnp.int32).reshape(_T, 1), ln_w.reshape(1, _D),
      w_router, tril)


def _gmm_body(tg_ref, act_ref, ng_ref, xg_ref, w1_ref, w2_ref, y_ref,
              acc_ref, xb_ref, w1b_ref, w2b_ref):
    f = pl.program_id(0)
    s = pl.program_id(1)

    @pl.when(act_ref[s] == 1)
    def _():
        @pl.when(f == 0)
        def _():
            xb_ref[pl.ds(s * _BM, _BM), :] = xg_ref[...].astype(jnp.bfloat16)

        @pl.when(ng_ref[s] == 1)
        def _():
            w1b_ref[...] = w1_ref[0].astype(jnp.bfloat16)
            w2b_ref[...] = w2_ref[0].astype(jnp.bfloat16)

        xb = xb_ref[pl.ds(s * _BM, _BM), :]
        h = jnp.dot(xb, w1b_ref[...], preferred_element_type=jnp.float32)
        h = jax.nn.gelu(h)
        contrib = jnp.dot(h.astype(jnp.bfloat16), w2b_ref[...],
                          preferred_element_type=jnp.float32)

        @pl.when(f == 0)
        def _():
            acc_ref[pl.ds(s * _BM, _BM), :] = contrib

        @pl.when((f > 0) & (f < _NF - 1))
        def _():
            acc_ref[pl.ds(s * _BM, _BM), :] += contrib

        @pl.when(f == _NF - 1)
        def _():
            y_ref[...] = acc_ref[pl.ds(s * _BM, _BM), :] + contrib


def _gmm(xg, w1, w2, tg, act, ng):
    grid_spec = pltpu.PrefetchScalarGridSpec(
        num_scalar_prefetch=3,
        grid=(_NF, _NT),
        in_specs=[
            pl.BlockSpec((_BM, _D),
                         lambda f, s, tg, act, ng: (jnp.where(f == 0, s, 0), 0)),
            pl.BlockSpec((1, _D, _FB), lambda f, s, tg, act, ng: (tg[s], 0, f)),
            pl.BlockSpec((1, _FB, _D), lambda f, s, tg, act, ng: (tg[s], f, 0)),
        ],
        out_specs=pl.BlockSpec(
            (_BM, _D),
            lambda f, s, tg, act, ng: (jnp.where(f == _NF - 1, s, 0), 0)),
        scratch_shapes=[
            pltpu.VMEM((_P, _D), jnp.float32),
            pltpu.VMEM((_P, _D), jnp.bfloat16),
            pltpu.VMEM((_D, _FB), jnp.bfloat16),
            pltpu.VMEM((_FB, _D), jnp.bfloat16),
        ],
    )
    return pl.pallas_call(
        _gmm_body,
        grid_spec=grid_spec,
        out_shape=jax.ShapeDtypeStruct((_P, _D), jnp.float32),
    )(tg, act, ng, xg, w1, w2)


_MESH = plsc.VectorSubcoreMesh(core_axis_name="c", subcore_axis_name="s")
_NC = 2
_NW = 32


def _wid():
    return jax.lax.axis_index("s") * _NC + jax.lax.axis_index("c")


@functools.partial(
    pl.kernel,
    out_type=jax.ShapeDtypeStruct((_P + _T * _E, ), jnp.int32),
    mesh=_MESH,
    scratch_types=[
        pltpu.VMEM((4, 128), jnp.int32),
        pltpu.VMEM((4, 128), jnp.int32),
        pltpu.SemaphoreType.DMA,
    ],
)
def _sc_scatter_tok(pos_hbm, tok_hbm, idx_v, val_v, sem):
    w = _wid()
    for j in range(4):
        pltpu.sync_copy(pos_hbm.at[pl.ds(w * 512 + j * 128, 128)], idx_v.at[j])
    lane = jax.lax.iota(jnp.int32, 16)
    for j in range(4):
        for k in range(8):
            g = (w * 4 + j) * 128 + k * 16
            val_v[j, pl.ds(k * 16, 16)] = (g + lane) >> 3
    copies = [pltpu.async_copy(val_v.at[j], tok_hbm.at[idx_v.at[j]], sem)
              for j in range(4)]
    for cp in copies:
        cp.wait()


@functools.partial(
    pl.kernel,
    out_type=jax.ShapeDtypeStruct((_P, _D), jnp.float32),
    mesh=_MESH,
    scratch_types=[
        pltpu.VMEM((10, 16), jnp.int32),
        pltpu.VMEM((16, _D), jnp.float32),
        pltpu.VMEM((16, _D), jnp.float32),
        pltpu.VMEM((16, _D), jnp.float32),
        pltpu.VMEM((16, _D), jnp.float32),
        pltpu.SemaphoreType.DMA,
        pltpu.SemaphoreType.DMA,
        pltpu.SemaphoreType.DMA,
        pltpu.SemaphoreType.DMA,
        pltpu.SemaphoreType.DMA,
        pltpu.SemaphoreType.DMA,
        pltpu.SemaphoreType.DMA,
        pltpu.SemaphoreType.DMA,
    ],
)
def _sc_gather_rows(tok_hbm, x_hbm, xg_hbm, idx_v, r0, r1, r2, r3,
                    gs0, gs1, gs2, gs3, ws0, ws1, ws2, ws3):
    w = _wid()
    for j in range(10):
        pltpu.sync_copy(tok_hbm.at[pl.ds(w * 160 + j * 16, 16)], idx_v.at[j])
    for j in range(10):
        v = idx_v[j, pl.ds(0, 16)]
        idx_v[j, pl.ds(0, 16)] = jnp.minimum(jnp.maximum(v, 0), _T - 1)
    bufs = [r0, r1, r2, r3]
    gsems = [gs0, gs1, gs2, gs3]
    wsems = [ws0, ws1, ws2, ws3]
    gcopies = [None] * 4
    wcopies = [None] * 4
    for c in range(4):
        gcopies[c] = pltpu.async_copy(x_hbm.at[idx_v.at[c]], bufs[c], gsems[c])
    for j in range(10):
        b = j % 4
        gcopies[b].wait()
        wcopies[b] = pltpu.async_copy(
            bufs[b], xg_hbm.at[pl.ds(w * 160 + j * 16, 16)], wsems[b])
        if j + 4 < 10:
            wcopies[b].wait()
            gcopies[b] = pltpu.async_copy(
                x_hbm.at[idx_v.at[j + 4]], bufs[b], gsems[b])
    for j in range(6, 10):
        wcopies[j % 4].wait()


@functools.partial(
    pl.kernel,
    out_type=jax.ShapeDtypeStruct((_T, _D), jnp.float32),
    mesh=_MESH,
    scratch_types=[
        pltpu.VMEM((4, 32), jnp.int32),
        pltpu.VMEM((4, 32), jnp.float32),
        pltpu.VMEM((16, _D), jnp.float32),
        pltpu.VMEM((32, _D), jnp.float32),
        pltpu.VMEM((32, _D), jnp.float32),
        pltpu.VMEM((16, _D), jnp.float32),
        pltpu.SemaphoreType.DMA,
        pltpu.SemaphoreType.DMA,
    ],
)
def _sc_combine(pos2_hbm, p2_hbm, x_hbm, y_hbm, out_hbm,
                idx_v, pv, xr, yr_a, yr_b, orows, sem_a, sem_b):
    w = _wid()
    for c in range(4):
        pltpu.sync_copy(pos2_hbm.at[pl.ds(w * 128 + c * 32, 32)], idx_v.at[c])
        pltpu.sync_copy(p2_hbm.at[pl.ds(w * 128 + c * 32, 32)], pv.at[c])
    ybufs = [(yr_a, sem_a), (yr_b, sem_b)]
    ycopies = [pltpu.async_copy(y_hbm.at[idx_v.at[c]], ybufs[c][0], ybufs[c][1])
               for c in range(2)]
    for c in range(4):
        ycopies[c % 2].wait()
        yr = ybufs[c % 2][0]
        pltpu.sync_copy(x_hbm.at[pl.ds(w * 64 + c * 16, 16)], xr)
        pva = pv[c, pl.ds(0, 16)]
        pvb = pv[c, pl.ds(16, 16)]
        ps = [pva[i] for i in range(16)] + [pvb[i] for i in range(16)]

        def body(q, _):
            for r in range(16):
                sl = pl.ds(q * 16, 16)
                orows[r, sl] = (xr[r, sl] + ps[2 * r] * yr[2 * r, sl]
                                + ps[2 * r + 1] * yr[2 * r + 1, sl])
            return 0

        jax.lax.fori_loop(0, _D // 16, body, 0)
        pltpu.sync_copy(orows, out_hbm.at[pl.ds(w * 64 + c * 16, 16)])
        if c + 2 < 4:
            ycopies[c % 2] = pltpu.async_copy(
                y_hbm.at[idx_v.at[c + 2]], ybufs[c % 2][0], ybufs[c % 2][1])


def kernel(x, mask, ln_w, w_router, w1, w2):
    tril = jnp.tri(_T, _T, -1, dtype=jnp.bfloat16)
    pos, pos2, p2, tg, act, ng = _dispatch(x, mask, ln_w, w_router, tril)
    tg = tg.reshape(_NT)
    act = act.reshape(_NT)
    ng = ng.reshape(_NT)

    tok_sorted = _sc_scatter_tok(pos.reshape(_T * _E))
    xg = _sc_gather_rows(tok_sorted[:_P], x)

    y = _gmm(xg, w1, w2, tg, act, ng)

    out = _sc_combine(pos2.reshape(_T * _K), p2.reshape(_T * _K), x, y)
    return out


# speedup vs baseline: 1.0567x; 1.0567x over previous
import functools

import jax
import jax.numpy as jnp
from jax.experimental import pallas as pl
from jax.experimental.pallas import tpu as pltpu
from jax.experimental.pallas import tpu_sc as plsc

_T = 2048
_D = 1024
_E = 8
_K = 2
_F = 4096
_EPS = 1e-5

_BM = 256
_NT = _T * _K // _BM + _E
_P = _NT * _BM
_FB = 512
_NF = _F // _FB

_NEG = jnp.finfo(jnp.float32).min


def _dispatch_body(x_ref, mask_ref, lnw_ref, wr_ref, tril_ref,
                   pos_ref, pos2_ref, p2_ref, tg_ref, act_ref, ng_ref):
    x = x_ref[...]
    var = jnp.mean(x * x, axis=-1, keepdims=True)
    xn = x * jax.lax.rsqrt(var + _EPS) * lnw_ref[...]
    logits = jnp.dot(xn, wr_ref[...], preferred_element_type=jnp.float32)
    mask = mask_ref[...] != 0
    logits = jnp.where(mask, logits, _NEG)

    ii = jax.lax.broadcasted_iota(jnp.int32, logits.shape, 1)
    m1 = jnp.max(logits, axis=-1, keepdims=True)
    i1 = jnp.min(jnp.where(logits == m1, ii, _E), axis=-1, keepdims=True)
    is1 = ii == i1
    l2 = jnp.where(is1, _NEG, logits)
    m2 = jnp.max(l2, axis=-1, keepdims=True)
    i2 = jnp.min(jnp.where(l2 == m2, ii, _E), axis=-1, keepdims=True)
    is2 = ii == i2
    hard = is1 | is2

    z = jnp.exp(logits - m1)
    probs = z / jnp.sum(z, axis=-1, keepdims=True)
    probs = jnp.where(hard & mask, probs, 0.0)

    sel = hard.astype(jnp.bfloat16)
    rank = jnp.dot(tril_ref[...], sel, preferred_element_type=jnp.float32)
    rank = rank.astype(jnp.int32)
    counts = jnp.sum(hard.astype(jnp.float32), axis=0, keepdims=True)
    counts = counts.astype(jnp.int32)
    c_pad = ((counts + _BM - 1) // _BM) * _BM
    ei = jax.lax.broadcasted_iota(jnp.int32, (_E, _E), 0)
    ej = jax.lax.broadcasted_iota(jnp.int32, (_E, _E), 1)
    upper = (ei < ej).astype(jnp.bfloat16)
    off = jnp.dot(c_pad.astype(jnp.bfloat16), upper,
                  preferred_element_type=jnp.float32).astype(jnp.int32)
    ends = off + c_pad

    pos = off + rank
    ti = jax.lax.broadcasted_iota(jnp.int32, logits.shape, 0)
    pos_ref[...] = jnp.where(hard, pos, _P + ti * _E + ii)

    pos1 = jnp.sum(jnp.where(is1, pos, 0), axis=-1, keepdims=True)
    pos2_ = jnp.sum(jnp.where(is2, pos, 0), axis=-1, keepdims=True)
    pos2_ref[...] = jnp.concatenate([pos1, pos2_], axis=-1)
    pb1 = jnp.sum(jnp.where(is1, probs, 0.0), axis=-1, keepdims=True)
    pb2 = jnp.sum(jnp.where(is2, probs, 0.0), axis=-1, keepdims=True)
    p2_ref[...] = jnp.concatenate([pb1, pb2], axis=-1)

    ms = jax.lax.broadcasted_iota(jnp.int32, (_NT, _E), 0) * _BM
    tg = jnp.sum((ms >= ends).astype(jnp.int32), axis=-1, keepdims=True)
    tg = jnp.minimum(tg, _E - 1)
    tg_ref[...] = tg
    mc = jax.lax.broadcasted_iota(jnp.int32, (_NT, 1), 0) * _BM
    act_ref[...] = (mc < ends[:, _E - 1:_E]).astype(jnp.int32)
    prev = jnp.concatenate([jnp.full((1, 1), -1, jnp.int32), tg[:-1]], axis=0)
    ng_ref[...] = (tg != prev).astype(jnp.int32)


def _dispatch(x, mask, ln_w, w_router, tril):
    return pl.pallas_call(
        _dispatch_body,
        out_shape=[
            jax.ShapeDtypeStruct((_T, _E), jnp.int32),
            jax.ShapeDtypeStruct((_T, _K), jnp.int32),
            jax.ShapeDtypeStruct((_T, _K), jnp.float32),
            jax.ShapeDtypeStruct((_NT, 1), jnp.int32),
            jax.ShapeDtypeStruct((_NT, 1), jnp.int32),
            jax.ShapeDtypeStruct((_NT, 1), jnp.int32),
        ],
    )(x, mask.astype(jnp.int32).reshape(_T, 1), ln_w.reshape(1, _D),
      w_router, tril)


def _gmm_body(tg_ref, act_ref, ng_ref, xg_ref, w1_ref, w2_ref, y_ref,
              acc_ref, xb_ref, w1b_ref, w2b_ref):
    f = pl.program_id(0)
    s = pl.program_id(1)

    @pl.when(act_ref[s] == 1)
    def _():
        @pl.when(f == 0)
        def _():
            xb_ref[pl.ds(s * _BM, _BM), :] = xg_ref[...].astype(jnp.bfloat16)

        @pl.when(ng_ref[s] == 1)
        def _():
            w1b_ref[...] = w1_ref[0].astype(jnp.bfloat16)
            w2b_ref[...] = w2_ref[0].astype(jnp.bfloat16)

        xb = xb_ref[pl.ds(s * _BM, _BM), :]
        h = jnp.dot(xb, w1b_ref[...], preferred_element_type=jnp.float32)
        h = jax.nn.gelu(h)
        contrib = jnp.dot(h.astype(jnp.bfloat16), w2b_ref[...],
                          preferred_element_type=jnp.float32)

        @pl.when(f == 0)
        def _():
            acc_ref[pl.ds(s * _BM, _BM), :] = contrib

        @pl.when((f > 0) & (f < _NF - 1))
        def _():
            acc_ref[pl.ds(s * _BM, _BM), :] += contrib

        @pl.when(f == _NF - 1)
        def _():
            y_ref[...] = acc_ref[pl.ds(s * _BM, _BM), :] + contrib


def _gmm(xg, w1, w2, tg, act, ng):
    grid_spec = pltpu.PrefetchScalarGridSpec(
        num_scalar_prefetch=3,
        grid=(_NF, _NT),
        in_specs=[
            pl.BlockSpec((_BM, _D),
                         lambda f, s, tg, act, ng: (jnp.where(f == 0, s, 0), 0)),
            pl.BlockSpec((1, _D, _FB), lambda f, s, tg, act, ng: (tg[s], 0, f)),
            pl.BlockSpec((1, _FB, _D), lambda f, s, tg, act, ng: (tg[s], f, 0)),
        ],
        out_specs=pl.BlockSpec(
            (_BM, _D),
            lambda f, s, tg, act, ng: (jnp.where(f == _NF - 1, s, 0), 0)),
        scratch_shapes=[
            pltpu.VMEM((_P, _D), jnp.float32),
            pltpu.VMEM((_P, _D), jnp.bfloat16),
            pltpu.VMEM((_D, _FB), jnp.bfloat16),
            pltpu.VMEM((_FB, _D), jnp.bfloat16),
        ],
    )
    return pl.pallas_call(
        _gmm_body,
        grid_spec=grid_spec,
        out_shape=jax.ShapeDtypeStruct((_P, _D), jnp.float32),
    )(tg, act, ng, xg, w1, w2)


_MESH = plsc.VectorSubcoreMesh(core_axis_name="c", subcore_axis_name="s")
_NC = 2
_NW = 32


def _wid():
    return jax.lax.axis_index("s") * _NC + jax.lax.axis_index("c")


@functools.partial(
    pl.kernel,
    out_type=jax.ShapeDtypeStruct((_P + _T * _E, ), jnp.int32),
    mesh=_MESH,
    scratch_types=[
        pltpu.VMEM((4, 128), jnp.int32),
        pltpu.VMEM((4, 128), jnp.int32),
        pltpu.SemaphoreType.DMA,
    ],
)
def _sc_scatter_tok(pos_hbm, tok_hbm, idx_v, val_v, sem):
    w = _wid()
    for j in range(4):
        pltpu.sync_copy(pos_hbm.at[pl.ds(w * 512 + j * 128, 128)], idx_v.at[j])
    lane = jax.lax.iota(jnp.int32, 16)
    for j in range(4):
        for k in range(8):
            g = (w * 4 + j) * 128 + k * 16
            val_v[j, pl.ds(k * 16, 16)] = (g + lane) >> 3
    copies = [pltpu.async_copy(val_v.at[j], tok_hbm.at[idx_v.at[j]], sem)
              for j in range(4)]
    for cp in copies:
        cp.wait()


@functools.partial(
    pl.kernel,
    out_type=jax.ShapeDtypeStruct((_P, _D), jnp.float32),
    mesh=_MESH,
    scratch_types=[
        pltpu.VMEM((_P // 512, 16), jnp.int32),
        pltpu.VMEM((16, _D), jnp.float32),
        pltpu.VMEM((16, _D), jnp.float32),
        pltpu.VMEM((16, _D), jnp.float32),
        pltpu.VMEM((16, _D), jnp.float32),
        pltpu.SemaphoreType.DMA,
        pltpu.SemaphoreType.DMA,
        pltpu.SemaphoreType.DMA,
        pltpu.SemaphoreType.DMA,
        pltpu.SemaphoreType.DMA,
        pltpu.SemaphoreType.DMA,
        pltpu.SemaphoreType.DMA,
        pltpu.SemaphoreType.DMA,
    ],
)
def _sc_gather_rows(tok_hbm, x_hbm, xg_hbm, idx_v, r0, r1, r2, r3,
                    gs0, gs1, gs2, gs3, ws0, ws1, ws2, ws3):
    w = _wid()
    nck = _P // 512
    rpw = _P // 32
    for j in range(nck):
        pltpu.sync_copy(tok_hbm.at[pl.ds(w * rpw + j * 16, 16)], idx_v.at[j])
    for j in range(nck):
        v = idx_v[j, pl.ds(0, 16)]
        idx_v[j, pl.ds(0, 16)] = jnp.minimum(jnp.maximum(v, 0), _T - 1)
    bufs = [r0, r1, r2, r3]
    gsems = [gs0, gs1, gs2, gs3]
    wsems = [ws0, ws1, ws2, ws3]
    gcopies = [None] * 4
    wcopies = [None] * 4
    for c in range(4):
        gcopies[c] = pltpu.async_copy(x_hbm.at[idx_v.at[c]], bufs[c], gsems[c])
    for j in range(nck):
        b = j % 4
        gcopies[b].wait()
        wcopies[b] = pltpu.async_copy(
            bufs[b], xg_hbm.at[pl.ds(w * rpw + j * 16, 16)], wsems[b])
        if j + 4 < nck:
            wcopies[b].wait()
            gcopies[b] = pltpu.async_copy(
                x_hbm.at[idx_v.at[j + 4]], bufs[b], gsems[b])
    for j in range(max(nck - 4, 0), nck):
        wcopies[j % 4].wait()


@functools.partial(
    pl.kernel,
    out_type=jax.ShapeDtypeStruct((_T, _D), jnp.float32),
    mesh=_MESH,
    scratch_types=[
        pltpu.VMEM((4, 32), jnp.int32),
        pltpu.VMEM((4, 32), jnp.float32),
        pltpu.VMEM((16, _D), jnp.float32),
        pltpu.VMEM((32, _D), jnp.float32),
        pltpu.VMEM((32, _D), jnp.float32),
        pltpu.VMEM((16, _D), jnp.float32),
        pltpu.SemaphoreType.DMA,
        pltpu.SemaphoreType.DMA,
    ],
)
def _sc_combine(pos2_hbm, p2_hbm, x_hbm, y_hbm, out_hbm,
                idx_v, pv, xr, yr_a, yr_b, orows, sem_a, sem_b):
    w = _wid()
    for c in range(4):
        pltpu.sync_copy(pos2_hbm.at[pl.ds(w * 128 + c * 32, 32)], idx_v.at[c])
        pltpu.sync_copy(p2_hbm.at[pl.ds(w * 128 + c * 32, 32)], pv.at[c])
    ybufs = [(yr_a, sem_a), (yr_b, sem_b)]
    ycopies = [pltpu.async_copy(y_hbm.at[idx_v.at[c]], ybufs[c][0], ybufs[c][1])
               for c in range(2)]
    for c in range(4):
        ycopies[c % 2].wait()
        yr = ybufs[c % 2][0]
        pltpu.sync_copy(x_hbm.at[pl.ds(w * 64 + c * 16, 16)], xr)
        pva = pv[c, pl.ds(0, 16)]
        pvb = pv[c, pl.ds(16, 16)]
        ps = [pva[i] for i in range(16)] + [pvb[i] for i in range(16)]

        def body(q, _):
            for r in range(16):
                sl = pl.ds(q * 16, 16)
                orows[r, sl] = (xr[r, sl] + ps[2 * r] * yr[2 * r, sl]
                                + ps[2 * r + 1] * yr[2 * r + 1, sl])
            return 0

        jax.lax.fori_loop(0, _D // 16, body, 0)
        pltpu.sync_copy(orows, out_hbm.at[pl.ds(w * 64 + c * 16, 16)])
        if c + 2 < 4:
            ycopies[c % 2] = pltpu.async_copy(
                y_hbm.at[idx_v.at[c + 2]], ybufs[c % 2][0], ybufs[c % 2][1])


def kernel(x, mask, ln_w, w_router, w1, w2):
    tril = jnp.tri(_T, _T, -1, dtype=jnp.bfloat16)
    pos, pos2, p2, tg, act, ng = _dispatch(x, mask, ln_w, w_router, tril)
    tg = tg.reshape(_NT)
    act = act.reshape(_NT)
    ng = ng.reshape(_NT)

    tok_sorted = _sc_scatter_tok(pos.reshape(_T * _E))
    xg = _sc_gather_rows(tok_sorted[:_P], x)

    y = _gmm(xg, w1, w2, tg, act, ng)

    out = _sc_combine(pos2.reshape(_T * _K), p2.reshape(_T * _K), x, y)
    return out
